# Initial kernel scaffold; baseline (speedup 1.0000x reference)
import jax
import jax.numpy as jnp
from jax.experimental import pallas as pl


def kernel(x, edge_index, batch, W1f, b1f, W1b, b1b, W2f, b2f, W2b, b2b):
    raise NotImplementedError("write your pallas kernel here")


# speedup vs baseline: 4.6162x; 4.6162x over previous
import functools
import numpy as np
import jax
import jax.numpy as jnp
from jax import lax
from jax.experimental import pallas as pl
from jax.experimental.pallas import tpu as pltpu
from jax.experimental.pallas import tpu_sc as plsc

N_NODES_C = 10000
NPAD = 10240
EPAD = 163840
NT = 16
CHUNKS = 80
C = 128
STRIPE = NPAD // NT
NGRAPH = 16
LAM = 3.0
KCH = 5


def _cheb_coeffs(k, lam):
    a = 2.0 / lam - 1.0
    b = -2.0 / lam
    cm = np.zeros((k, k), np.float64)
    cm[0, 0] = 1.0
    cm[1, 0] = a
    cm[1, 1] = b
    for i in range(2, k):
        cm[i] = 2.0 * (a * cm[i - 1] + b * np.roll(cm[i - 1], 1)) - cm[i - 2]
    return cm


def _make_spmm(s_slabs, t_rows):
    mesh = plsc.VectorSubcoreMesh(core_axis_name="c", subcore_axis_name="s")

    @functools.partial(
        pl.kernel,
        mesh=mesh,
        out_type=jax.ShapeDtypeStruct((s_slabs * NPAD, 128), jnp.float32),
        scratch_types=[
            pltpu.VMEM((CHUNKS, C), jnp.int32),
            pltpu.VMEM((CHUNKS, C), jnp.int32),
            pltpu.VMEM((C, 128), jnp.float32),
            pltpu.VMEM_SHARED((NPAD, 128), jnp.float32),
            pltpu.SemaphoreType.DMA,
        ],
    )
    def k(tab, cols, rows, zstripe, out, colv, rowv, gbuf, acc, gsem):
        cid = lax.axis_index("c")
        tid = lax.axis_index("s")
        for si in range(s_slabs // 2):
            slab = 2 * si + cid
            pltpu.sync_copy(zstripe, acc.at[pl.ds(tid * STRIPE, STRIPE), :])
            pltpu.sync_copy(cols.at[slab, tid], colv)
            pltpu.sync_copy(rows.at[slab, tid], rowv)
            plsc.subcore_barrier()

            def chunk(kk, carry):
                pltpu.async_copy(tab.at[colv.at[kk]], gbuf, gsem).wait()
                pltpu.sync_copy(gbuf, acc.at[rowv.at[kk]], add=True)
                return carry

            lax.fori_loop(0, CHUNKS, chunk, 0)
            plsc.subcore_barrier()
            pltpu.sync_copy(
                acc.at[pl.ds(tid * STRIPE, STRIPE), :],
                out.at[pl.ds(slab * NPAD + tid * STRIPE, STRIPE), :],
            )
            plsc.subcore_barrier()

    return k


def _dinv_kernel(deg2):
    def body(df_ref, db_ref, o1, o2, o3, o4):
        for dref, oa, ob in ((df_ref, o1, o2), (db_ref, o3, o4)):
            deg = dref[...]
            dinv = jnp.where(
                deg > 0, lax.rsqrt(jnp.maximum(deg, 1e-12)), 0.0
            )
            oa[...] = dinv
            ob[...] = dinv * dinv

    nb = NPAD // 256
    spec = pl.BlockSpec((256, 128), lambda i: (i, 0))
    outs = [jax.ShapeDtypeStruct((NPAD, 128), jnp.float32)] * 4
    return pl.pallas_call(
        body,
        grid=(nb,),
        in_specs=[
            pl.BlockSpec((256, 128), lambda i: (i, 0)),
            pl.BlockSpec((256, 128), lambda i: (i, 0)),
        ],
        out_specs=[spec] * 4,
        out_shape=outs,
    )(deg2[:NPAD], deg2[NPAD:])


def _scale_rowmajor(x, scale, s_slabs):
    def body(x_ref, sc_ref, o_ref):
        o_ref[...] = x_ref[...] * sc_ref[...]

    return pl.pallas_call(
        body,
        grid=(s_slabs, NPAD // 256),
        in_specs=[
            pl.BlockSpec((256, 128), lambda s, i: (i, s)),
            pl.BlockSpec((256, 128), lambda s, i: (i, 0)),
        ],
        out_specs=pl.BlockSpec((256, 128), lambda s, i: (s * (NPAD // 256) + i, 0)),
        out_shape=jax.ShapeDtypeStruct((s_slabs * NPAD, 128), jnp.float32),
    )(x, scale)


def _scale_slabmajor(y, scale, s_slabs):
    def body(y_ref, sc_ref, o_ref):
        o_ref[...] = y_ref[...] * sc_ref[...]

    nb = NPAD // 256
    return pl.pallas_call(
        body,
        grid=(s_slabs, nb),
        in_specs=[
            pl.BlockSpec((256, 128), lambda s, i: (s * nb + i, 0)),
            pl.BlockSpec((256, 128), lambda s, i: (i, 0)),
        ],
        out_specs=pl.BlockSpec((256, 128), lambda s, i: (s * nb + i, 0)),
        out_shape=jax.ShapeDtypeStruct((s_slabs * NPAD, 128), jnp.float32),
    )(y, scale)


def _layer_matmul(x_in, yfs, ybs, w0, wf, wb, bias, dinvf, dinvb, s_in):
    dx = x_in.shape[1]
    nb = NPAD // 256

    def body(x_ref, yf0, yf1, yf2, yf3, yb0, yb1, yb2, yb3,
             w0_ref, wf_ref, wb_ref, b_ref, df_ref, db_ref, o_ref):
        yf_refs = (yf0, yf1, yf2, yf3)
        yb_refs = (yb0, yb1, yb2, yb3)
        acc = jnp.dot(x_ref[...], w0_ref[...],
                      preferred_element_type=jnp.float32)
        accf = jnp.zeros((256, 512), jnp.float32)
        accb = jnp.zeros((256, 512), jnp.float32)
        for j in range(4):
            for s in range(s_in):
                accf += jnp.dot(yf_refs[j][s], wf_ref[j, s],
                                preferred_element_type=jnp.float32)
                accb += jnp.dot(yb_refs[j][s], wb_ref[j, s],
                                preferred_element_type=jnp.float32)
        df = jnp.concatenate([df_ref[...]] * 4, axis=1)
        db = jnp.concatenate([db_ref[...]] * 4, axis=1)
        out = acc + b_ref[0:1, :] + df * accf + db * accb
        o_ref[...] = jnp.maximum(out, 0.0)

    y_spec = pl.BlockSpec((s_in, 256, 128), lambda i: (0, i, 0))
    return pl.pallas_call(
        body,
        grid=(nb,),
        in_specs=[
            pl.BlockSpec((256, dx), lambda i: (i, 0)),
            y_spec, y_spec, y_spec, y_spec,
            y_spec, y_spec, y_spec, y_spec,
            pl.BlockSpec((dx, 512), lambda i: (0, 0)),
            pl.BlockSpec((4, s_in, 128, 512), lambda i: (0, 0, 0, 0)),
            pl.BlockSpec((4, s_in, 128, 512), lambda i: (0, 0, 0, 0)),
            pl.BlockSpec((8, 512), lambda i: (0, 0)),
            pl.BlockSpec((256, 128), lambda i: (i, 0)),
            pl.BlockSpec((256, 128), lambda i: (i, 0)),
        ],
        out_specs=pl.BlockSpec((256, 512), lambda i: (i, 0)),
        out_shape=jax.ShapeDtypeStruct((NPAD, 512), jnp.float32),
    )(x_in, *yfs, *ybs, w0, wf, wb, bias, dinvf, dinvb)


def _pool(h, oht):
    nb = NPAD // 256

    def body(oh_ref, h_ref, o_ref, acc, cnt):
        i = pl.program_id(0)

        @pl.when(i == 0)
        def _():
            acc[...] = jnp.zeros_like(acc)
            cnt[...] = jnp.zeros_like(cnt)

        oh = oh_ref[...]
        acc[...] += jnp.dot(oh, h_ref[...], preferred_element_type=jnp.float32)
        cnt[...] += jnp.sum(oh, axis=1, keepdims=True)

        @pl.when(i == nb - 1)
        def _():
            o_ref[...] = acc[...] / jnp.maximum(cnt[...][:, 0:1], 1.0)

    return pl.pallas_call(
        body,
        grid=(nb,),
        in_specs=[
            pl.BlockSpec((16, 256), lambda i: (0, i)),
            pl.BlockSpec((256, 512), lambda i: (i, 0)),
        ],
        out_specs=pl.BlockSpec((16, 512), lambda i: (0, 0)),
        out_shape=jax.ShapeDtypeStruct((16, 512), jnp.float32),
        scratch_shapes=[
            pltpu.VMEM((16, 512), jnp.float32),
            pltpu.VMEM((16, 128), jnp.float32),
        ],
    )(oht, h)


def _edge_streams(idx, n_edges):
    extra = EPAD - n_edges
    pad_g = jnp.arange(extra, dtype=jnp.int32) % N_NODES_C
    pad_s = N_NODES_C + (jnp.arange(extra, dtype=jnp.int32) % (NPAD - N_NODES_C))
    g = jnp.concatenate([idx, pad_g]).reshape(NT, CHUNKS, C)
    s = jnp.concatenate([idx, pad_s]).reshape(NT, CHUNKS, C)
    return g, s


def _chain(x2d, dinv, dinv2, cols, rows, s_slabs, spmm):
    u = _scale_rowmajor(x2d, dinv, s_slabs)
    ys = []
    for j in range(1, KCH):
        y = spmm(u, cols, rows)
        ys.append(y.reshape(s_slabs, NPAD, 128))
        if j < KCH - 1:
            u = _scale_slabmajor(y, dinv2, s_slabs)
    return ys


def kernel(x, edge_index, batch, W1f, b1f, W1b, b1b, W2f, b2f, W2b, b2b):
    n, d_in = x.shape
    n_edges = edge_index.shape[1]
    cm = jnp.asarray(_cheb_coeffs(KCH, LAM), jnp.float32)

    src = edge_index[0].astype(jnp.int32)
    dst = edge_index[1].astype(jnp.int32)

    gsrc, ssrc = _edge_streams(src, n_edges)
    gdst, sdst = _edge_streams(dst, n_edges)

    def with_offsets(g, s_slabs):
        offs = (jnp.arange(s_slabs, dtype=jnp.int32) * NPAD)[:, None, None, None]
        return g[None] + offs

    rows_f = {s: jnp.broadcast_to(ssrc, (s,) + ssrc.shape) for s in (2, 4)}
    rows_b = {s: jnp.broadcast_to(sdst, (s,) + sdst.shape) for s in (2, 4)}
    cols_f = {s: with_offsets(gdst, s) for s in (2, 4)}
    cols_b = {s: with_offsets(gsrc, s) for s in (2, 4)}

    zeros_stripe = jnp.zeros((STRIPE, 128), jnp.float32)
    spmm1 = _make_spmm(2, 2 * NPAD)
    spmm2 = _make_spmm(4, 4 * NPAD)
    spmm_deg = _make_spmm(2, NPAD)

    ones_tab = jnp.ones((NPAD, 128), jnp.float32)
    deg_cols = jnp.stack([gsrc, gdst])
    deg_rows = jnp.stack([ssrc, sdst])
    deg2 = spmm_deg(ones_tab, deg_cols, deg_rows, zeros_stripe)
    dinvf, dinv2f, dinvb, dinv2b = _dinv_kernel(deg2)

    def prep_w(wf, wb, bf, bb, s_in):
        wpf = jnp.einsum("kj,kio->jio", cm, wf)
        wpb = jnp.einsum("kj,kio->jio", cm, wb)
        w0 = wpf[0] + wpb[0]
        wfs = wpf[1:].reshape(4, s_in, 128, 512)
        wbs = wpb[1:].reshape(4, s_in, 128, 512)
        bias = jnp.broadcast_to((bf + bb).reshape(1, 512), (8, 512))
        return w0, wfs, wbs, bias

    w0_1, wf_1, wb_1, bias_1 = prep_w(W1f, W1b, b1f, b1b, d_in // 128)
    w0_2, wf_2, wb_2, bias_2 = prep_w(W2f, W2b, b2f, b2b, 4)

    x_pad = jnp.pad(x, ((0, NPAD - n), (0, 0)))

    def spmm1_call(u, cols, rows):
        return spmm1(u, cols, rows, zeros_stripe)

    def spmm2_call(u, cols, rows):
        return spmm2(u, cols, rows, zeros_stripe)

    s1 = d_in // 128
    yf1 = _chain(x_pad, dinvf, dinv2f, cols_f[s1], rows_f[s1], s1, spmm1_call)
    yb1 = _chain(x_pad, dinvb, dinv2b, cols_b[s1], rows_b[s1], s1, spmm1_call)
    h1 = _layer_matmul(x_pad, yf1, yb1, w0_1, wf_1, wb_1, bias_1,
                       dinvf, dinvb, s1)

    yf2 = _chain(h1, dinvf, dinv2f, cols_f[4], rows_f[4], 4, spmm2_call)
    yb2 = _chain(h1, dinvb, dinv2b, cols_b[4], rows_b[4], 4, spmm2_call)
    h2 = _layer_matmul(h1, yf2, yb2, w0_2, wf_2, wb_2, bias_2,
                       dinvf, dinvb, 4)

    batch_pad = jnp.pad(batch.astype(jnp.int32), (0, NPAD - n),
                        constant_values=-1)
    oht = (batch_pad[None, :] == jnp.arange(NGRAPH, dtype=jnp.int32)[:, None]
           ).astype(jnp.float32)
    return _pool(h2, oht)


# speedup vs baseline: 4.9020x; 1.0619x over previous
import functools
import numpy as np
import jax
import jax.numpy as jnp
from jax import lax
from jax.experimental import pallas as pl
from jax.experimental.pallas import tpu as pltpu
from jax.experimental.pallas import tpu_sc as plsc

N_NODES_C = 10000
NPAD = 10240
EPAD = 163840
NT = 16
CHUNKS = 80
CL = 128
GRP = 8
STRIPE = NPAD // NT
NB = NPAD // 256
NGRAPH = 16
LAM = 3.0
KCH = 5


def _cheb_coeffs(k, lam):
    a = 2.0 / lam - 1.0
    b = -2.0 / lam
    cm = np.zeros((k, k), np.float64)
    cm[0, 0] = 1.0
    cm[1, 0] = a
    cm[1, 1] = b
    for i in range(2, k):
        cm[i] = 2.0 * (a * cm[i - 1] + b * np.roll(cm[i - 1], 1)) - cm[i - 2]
    return cm


def _make_spmm(ns):
    mesh = plsc.VectorSubcoreMesh(core_axis_name="c", subcore_axis_name="s")

    @functools.partial(
        pl.kernel,
        mesh=mesh,
        out_type=jax.ShapeDtypeStruct((ns * NPAD, 128), jnp.float32),
        scratch_types=[
            pltpu.VMEM((GRP, CL), jnp.int32),
            pltpu.VMEM((GRP, CL), jnp.int32),
            pltpu.VMEM((CL, 128), jnp.float32),
            pltpu.VMEM((CL, 128), jnp.float32),
            pltpu.VMEM_SHARED((NPAD, 128), jnp.float32),
            pltpu.SemaphoreType.DMA,
            pltpu.SemaphoreType.DMA,
        ],
    )
    def k(tab, cols, rows, zstripe, out, colv, rowv, gb0, gb1, acc, sem0, sem1):
        cid = lax.axis_index("c")
        tid = lax.axis_index("s")
        gbufs = (gb0, gb1)
        sems = (sem0, sem1)
        for si in range(ns // 2):
            slab = 2 * si + cid
            pltpu.sync_copy(zstripe, acc.at[pl.ds(tid * STRIPE, STRIPE), :])
            plsc.subcore_barrier()

            def group(g, carry):
                pltpu.sync_copy(cols.at[slab, tid, pl.ds(g * GRP, GRP)], colv)
                pltpu.sync_copy(rows.at[slab, tid, pl.ds(g * GRP, GRP)], rowv)
                handles = [None, None]
                handles[0] = pltpu.async_copy(tab.at[colv.at[0]], gb0, sem0)
                for kk in range(GRP):
                    b = kk % 2
                    if kk + 1 < GRP:
                        handles[1 - b] = pltpu.async_copy(
                            tab.at[colv.at[kk + 1]], gbufs[1 - b], sems[1 - b])
                    handles[b].wait()
                    pltpu.sync_copy(gbufs[b], acc.at[rowv.at[kk]], add=True)
                return carry

            lax.fori_loop(0, CHUNKS // GRP, group, 0)
            plsc.subcore_barrier()
            pltpu.sync_copy(
                acc.at[pl.ds(tid * STRIPE, STRIPE), :],
                out.at[pl.ds(slab * NPAD + tid * STRIPE, STRIPE), :],
            )
            plsc.subcore_barrier()

    return k


def _dinv_kernel(deg2):
    def body(d_ref, o1, o2):
        deg = d_ref[...]
        dinv = jnp.where(deg > 0, lax.rsqrt(jnp.maximum(deg, 1e-12)), 0.0)
        o1[...] = dinv[None]
        o2[...] = (dinv * dinv)[None]

    return pl.pallas_call(
        body,
        grid=(2, NB),
        in_specs=[pl.BlockSpec((256, 128), lambda d, i: (d * NB + i, 0))],
        out_specs=[pl.BlockSpec((1, 256, 128), lambda d, i: (d, i, 0))] * 2,
        out_shape=[jax.ShapeDtypeStruct((2, NPAD, 128), jnp.float32)] * 2,
    )(deg2)


def _scale_rowmajor2(x, dscale, s_slabs):
    def body(x_ref, sc_ref, o_ref):
        o_ref[...] = x_ref[...] * sc_ref[0]

    return pl.pallas_call(
        body,
        grid=(2, s_slabs, NB),
        in_specs=[
            pl.BlockSpec((256, 128), lambda d, s, i: (i, s)),
            pl.BlockSpec((1, 256, 128), lambda d, s, i: (d, i, 0)),
        ],
        out_specs=pl.BlockSpec(
            (256, 128), lambda d, s, i: ((d * s_slabs + s) * NB + i, 0)),
        out_shape=jax.ShapeDtypeStruct((2 * s_slabs * NPAD, 128), jnp.float32),
    )(x, dscale)


def _scale_slabmajor2(y, dscale, s_slabs):
    def body(y_ref, sc_ref, o_ref):
        o_ref[...] = y_ref[...] * sc_ref[0]

    spec = pl.BlockSpec((256, 128), lambda d, s, i: ((d * s_slabs + s) * NB + i, 0))
    return pl.pallas_call(
        body,
        grid=(2, s_slabs, NB),
        in_specs=[
            spec,
            pl.BlockSpec((1, 256, 128), lambda d, s, i: (d, i, 0)),
        ],
        out_specs=spec,
        out_shape=jax.ShapeDtypeStruct((2 * s_slabs * NPAD, 128), jnp.float32),
    )(y, dscale)


def _layer_matmul(x_in, ys, w0, wf, wb, bias, dinv01, s_in):
    dx = x_in.shape[1]

    def body(x_ref, y0, y1, y2, y3, w0_ref, wf_ref, wb_ref, b_ref,
             d_ref, o_ref):
        y_refs = (y0, y1, y2, y3)
        acc = jnp.dot(x_ref[...], w0_ref[...],
                      preferred_element_type=jnp.float32)
        accf = jnp.zeros((256, 512), jnp.float32)
        accb = jnp.zeros((256, 512), jnp.float32)
        for j in range(4):
            for s in range(s_in):
                accf += jnp.dot(y_refs[j][s], wf_ref[j, s],
                                preferred_element_type=jnp.float32)
                accb += jnp.dot(y_refs[j][s_in + s], wb_ref[j, s],
                                preferred_element_type=jnp.float32)
        df = jnp.concatenate([d_ref[0]] * 4, axis=1)
        db = jnp.concatenate([d_ref[1]] * 4, axis=1)
        out = acc + b_ref[0:1, :] + df * accf + db * accb
        o_ref[...] = jnp.maximum(out, 0.0)

    y_spec = pl.BlockSpec((2 * s_in, 256, 128), lambda i: (0, i, 0))
    return pl.pallas_call(
        body,
        grid=(NB,),
        in_specs=[
            pl.BlockSpec((256, dx), lambda i: (i, 0)),
            y_spec, y_spec, y_spec, y_spec,
            pl.BlockSpec((dx, 512), lambda i: (0, 0)),
            pl.BlockSpec((4, s_in, 128, 512), lambda i: (0, 0, 0, 0)),
            pl.BlockSpec((4, s_in, 128, 512), lambda i: (0, 0, 0, 0)),
            pl.BlockSpec((8, 512), lambda i: (0, 0)),
            pl.BlockSpec((2, 256, 128), lambda i: (0, i, 0)),
        ],
        out_specs=pl.BlockSpec((256, 512), lambda i: (i, 0)),
        out_shape=jax.ShapeDtypeStruct((NPAD, 512), jnp.float32),
    )(x_in, *ys, w0, wf, wb, bias, dinv01)


def _pool(h, oht):
    def body(oh_ref, h_ref, o_ref, acc, cnt):
        i = pl.program_id(0)

        @pl.when(i == 0)
        def _():
            acc[...] = jnp.zeros_like(acc)
            cnt[...] = jnp.zeros_like(cnt)

        oh = oh_ref[...]
        acc[...] += jnp.dot(oh, h_ref[...], preferred_element_type=jnp.float32)
        cnt[...] += jnp.sum(oh, axis=1, keepdims=True)

        @pl.when(i == NB - 1)
        def _():
            o_ref[...] = acc[...] / jnp.maximum(cnt[...][:, 0:1], 1.0)

    return pl.pallas_call(
        body,
        grid=(NB,),
        in_specs=[
            pl.BlockSpec((16, 256), lambda i: (0, i)),
            pl.BlockSpec((256, 512), lambda i: (i, 0)),
        ],
        out_specs=pl.BlockSpec((16, 512), lambda i: (0, 0)),
        out_shape=jax.ShapeDtypeStruct((16, 512), jnp.float32),
        scratch_shapes=[
            pltpu.VMEM((16, 512), jnp.float32),
            pltpu.VMEM((16, 128), jnp.float32),
        ],
    )(oht, h)


def _edge_streams(idx, n_edges):
    extra = EPAD - n_edges
    pad_g = jnp.arange(extra, dtype=jnp.int32) % N_NODES_C
    pad_s = N_NODES_C + (jnp.arange(extra, dtype=jnp.int32) % (NPAD - N_NODES_C))
    shp = (NT, CHUNKS, CL)
    g = jnp.concatenate([idx, pad_g]).reshape(shp)
    s = jnp.concatenate([idx, pad_s]).reshape(shp)
    return g, s


def kernel(x, edge_index, batch, W1f, b1f, W1b, b1b, W2f, b2f, W2b, b2b):
    n, d_in = x.shape
    n_edges = edge_index.shape[1]
    cm = jnp.asarray(_cheb_coeffs(KCH, LAM), jnp.float32)

    src = edge_index[0].astype(jnp.int32)
    dst = edge_index[1].astype(jnp.int32)

    gsrc, ssrc = _edge_streams(src, n_edges)
    gdst, sdst = _edge_streams(dst, n_edges)

    def fused_idx(s_slabs):
        cols = jnp.stack(
            [gdst + (s * NPAD) for s in range(s_slabs)]
            + [gsrc + ((s_slabs + s) * NPAD) for s in range(s_slabs)])
        rows = jnp.stack([ssrc] * s_slabs + [sdst] * s_slabs)
        return cols, rows

    cols1, rows1 = fused_idx(d_in // 128)
    cols2, rows2 = fused_idx(4)

    zeros_stripe = jnp.zeros((STRIPE, 128), jnp.float32)
    spmm1 = _make_spmm(2 * (d_in // 128))
    spmm2 = _make_spmm(8)
    spmm_deg = _make_spmm(2)

    ones_tab = jnp.ones((NPAD, 128), jnp.float32)
    deg2 = spmm_deg(ones_tab, jnp.stack([gsrc, gdst]),
                    jnp.stack([ssrc, sdst]), zeros_stripe)
    dinv01, dinv201 = _dinv_kernel(deg2)

    def prep_w(wf, wb, bf, bb, s_in):
        wpf = jnp.einsum("kj,kio->jio", cm, wf)
        wpb = jnp.einsum("kj,kio->jio", cm, wb)
        w0 = wpf[0] + wpb[0]
        wfs = wpf[1:].reshape(4, s_in, 128, 512)
        wbs = wpb[1:].reshape(4, s_in, 128, 512)
        bias = jnp.broadcast_to((bf + bb).reshape(1, 512), (8, 512))
        return w0, wfs, wbs, bias

    w0_1, wf_1, wb_1, bias_1 = prep_w(W1f, W1b, b1f, b1b, d_in // 128)
    w0_2, wf_2, wb_2, bias_2 = prep_w(W2f, W2b, b2f, b2b, 4)

    x_pad = jnp.pad(x, ((0, NPAD - n), (0, 0)))

    def chain(x2d, cols, rows, s_in, spmm):
        u = _scale_rowmajor2(x2d, dinv01, s_in)
        ys = []
        for j in range(1, KCH):
            y = spmm(u, cols, rows, zeros_stripe)
            ys.append(y.reshape(2 * s_in, NPAD, 128))
            if j < KCH - 1:
                u = _scale_slabmajor2(y, dinv201, s_in)
        return ys

    s1 = d_in // 128
    ys1 = chain(x_pad, cols1, rows1, s1, spmm1)
    h1 = _layer_matmul(x_pad, ys1, w0_1, wf_1, wb_1, bias_1, dinv01, s1)

    ys2 = chain(h1, cols2, rows2, 4, spmm2)
    h2 = _layer_matmul(h1, ys2, w0_2, wf_2, wb_2, bias_2, dinv01, 4)

    batch_pad = jnp.pad(batch.astype(jnp.int32), (0, NPAD - n),
                        constant_values=-1)
    oht = (batch_pad[None, :] == jnp.arange(NGRAPH, dtype=jnp.int32)[:, None]
           ).astype(jnp.float32)
    return _pool(h2, oht)


# speedup vs baseline: 5.4177x; 1.1052x over previous
import functools
import numpy as np
import jax
import jax.numpy as jnp
from jax import lax
from jax.experimental import pallas as pl
from jax.experimental.pallas import tpu as pltpu
from jax.experimental.pallas import tpu_sc as plsc

N_NODES_C = 10000
NPAD = 10240
EPAD = 163840
NT = 16
CHUNKS = 80
CL = 128
GRP = 16
STRIPE = NPAD // NT
NB = NPAD // 256
NGRAPH = 16
LAM = 3.0
KCH = 5


def _cheb_coeffs(k, lam):
    a = 2.0 / lam - 1.0
    b = -2.0 / lam
    cm = np.zeros((k, k), np.float64)
    cm[0, 0] = 1.0
    cm[1, 0] = a
    cm[1, 1] = b
    for i in range(2, k):
        cm[i] = 2.0 * (a * cm[i - 1] + b * np.roll(cm[i - 1], 1)) - cm[i - 2]
    return cm


def _make_spmm(ns):
    mesh = plsc.VectorSubcoreMesh(core_axis_name="c", subcore_axis_name="s")

    @functools.partial(
        pl.kernel,
        mesh=mesh,
        out_type=jax.ShapeDtypeStruct((ns * NPAD, 128), jnp.float32),
        scratch_types=[
            pltpu.VMEM((2, GRP, CL), jnp.int32),
            pltpu.VMEM((2, GRP, CL), jnp.int32),
            pltpu.VMEM((CL, 128), jnp.float32),
            pltpu.VMEM((CL, 128), jnp.float32),
            pltpu.VMEM_SHARED((NPAD, 128), jnp.float32),
            pltpu.SemaphoreType.DMA,
            pltpu.SemaphoreType.DMA,
            pltpu.SemaphoreType.DMA((2,)),
            pltpu.SemaphoreType.DMA((2,)),
        ],
    )
    def k(tab, cols, rows, zstripe, out, colv, rowv, gb0, gb1, acc,
          sem0, sem1, csem, rsem):
        cid = lax.axis_index("c")
        tid = lax.axis_index("s")
        gbufs = (gb0, gb1)
        sems = (sem0, sem1)
        groups = CHUNKS // GRP

        def idx_start(slab, g, ib):
            pltpu.async_copy(
                cols.at[slab, tid, pl.ds(g * GRP, GRP)], colv.at[ib],
                csem.at[ib])
            pltpu.async_copy(
                rows.at[slab, tid, pl.ds(g * GRP, GRP)], rowv.at[ib],
                rsem.at[ib])

        def idx_wait(slab, g, ib):
            pltpu.make_async_copy(
                cols.at[slab, tid, pl.ds(g * GRP, GRP)], colv.at[ib],
                csem.at[ib]).wait()
            pltpu.make_async_copy(
                rows.at[slab, tid, pl.ds(g * GRP, GRP)], rowv.at[ib],
                rsem.at[ib]).wait()

        for si in range(ns // 2):
            slab = 2 * si + cid
            idx_start(slab, 0, 0)
            pltpu.sync_copy(zstripe, acc.at[pl.ds(tid * STRIPE, STRIPE), :])
            plsc.subcore_barrier()

            def group(g, carry):
                ib = lax.rem(g, 2)
                idx_wait(slab, g, ib)
                gn = jnp.minimum(g + 1, groups - 1)
                idx_start(slab, gn, 1 - ib)
                handles = [None, None]
                handles[0] = pltpu.async_copy(
                    tab.at[colv.at[ib, 0]], gb0, sem0)
                for kk in range(GRP):
                    b = kk % 2
                    if kk + 1 < GRP:
                        handles[1 - b] = pltpu.async_copy(
                            tab.at[colv.at[ib, kk + 1]],
                            gbufs[1 - b], sems[1 - b])
                    handles[b].wait()
                    pltpu.sync_copy(gbufs[b], acc.at[rowv.at[ib, kk]],
                                    add=True)
                return carry

            lax.fori_loop(0, groups, group, 0)
            idx_wait(slab, groups - 1, groups % 2)
            plsc.subcore_barrier()
            pltpu.sync_copy(
                acc.at[pl.ds(tid * STRIPE, STRIPE), :],
                out.at[pl.ds(slab * NPAD + tid * STRIPE, STRIPE), :],
            )
            plsc.subcore_barrier()

    return k


def _dinv_kernel(deg2):
    def body(d_ref, o1, o2):
        deg = d_ref[...]
        dinv = jnp.where(deg > 0, lax.rsqrt(jnp.maximum(deg, 1e-12)), 0.0)
        o1[...] = dinv[None]
        o2[...] = (dinv * dinv)[None]

    return pl.pallas_call(
        body,
        grid=(2, NB),
        in_specs=[pl.BlockSpec((256, 128), lambda d, i: (d * NB + i, 0))],
        out_specs=[pl.BlockSpec((1, 256, 128), lambda d, i: (d, i, 0))] * 2,
        out_shape=[jax.ShapeDtypeStruct((2, NPAD, 128), jnp.float32)] * 2,
    )(deg2)


def _scale_rowmajor2(x, dscale, s_slabs):
    def body(x_ref, sc_ref, o_ref):
        o_ref[...] = x_ref[...] * sc_ref[0]

    return pl.pallas_call(
        body,
        grid=(2, s_slabs, NB),
        in_specs=[
            pl.BlockSpec((256, 128), lambda d, s, i: (i, s)),
            pl.BlockSpec((1, 256, 128), lambda d, s, i: (d, i, 0)),
        ],
        out_specs=pl.BlockSpec(
            (256, 128), lambda d, s, i: ((d * s_slabs + s) * NB + i, 0)),
        out_shape=jax.ShapeDtypeStruct((2 * s_slabs * NPAD, 128), jnp.float32),
    )(x, dscale)


def _scale_slabmajor2(y, dscale, s_slabs):
    def body(y_ref, sc_ref, o_ref):
        o_ref[...] = y_ref[...] * sc_ref[0]

    spec = pl.BlockSpec((256, 128), lambda d, s, i: ((d * s_slabs + s) * NB + i, 0))
    return pl.pallas_call(
        body,
        grid=(2, s_slabs, NB),
        in_specs=[
            spec,
            pl.BlockSpec((1, 256, 128), lambda d, s, i: (d, i, 0)),
        ],
        out_specs=spec,
        out_shape=jax.ShapeDtypeStruct((2 * s_slabs * NPAD, 128), jnp.float32),
    )(y, dscale)


def _layer_matmul(x_in, ys, w0, wf, wb, bias, dinv01, s_in):
    dx = x_in.shape[1]

    def body(x_ref, y0, y1, y2, y3, w0_ref, wf_ref, wb_ref, b_ref,
             d_ref, o_ref):
        y_refs = (y0, y1, y2, y3)
        acc = jnp.dot(x_ref[...], w0_ref[...],
                      preferred_element_type=jnp.float32)
        accf = jnp.zeros((256, 512), jnp.float32)
        accb = jnp.zeros((256, 512), jnp.float32)
        for j in range(4):
            for s in range(s_in):
                accf += jnp.dot(y_refs[j][s], wf_ref[j, s],
                                preferred_element_type=jnp.float32)
                accb += jnp.dot(y_refs[j][s_in + s], wb_ref[j, s],
                                preferred_element_type=jnp.float32)
        df = jnp.concatenate([d_ref[0]] * 4, axis=1)
        db = jnp.concatenate([d_ref[1]] * 4, axis=1)
        out = acc + b_ref[0:1, :] + df * accf + db * accb
        o_ref[...] = jnp.maximum(out, 0.0)

    y_spec = pl.BlockSpec((2 * s_in, 256, 128), lambda i: (0, i, 0))
    return pl.pallas_call(
        body,
        grid=(NB,),
        in_specs=[
            pl.BlockSpec((256, dx), lambda i: (i, 0)),
            y_spec, y_spec, y_spec, y_spec,
            pl.BlockSpec((dx, 512), lambda i: (0, 0)),
            pl.BlockSpec((4, s_in, 128, 512), lambda i: (0, 0, 0, 0)),
            pl.BlockSpec((4, s_in, 128, 512), lambda i: (0, 0, 0, 0)),
            pl.BlockSpec((8, 512), lambda i: (0, 0)),
            pl.BlockSpec((2, 256, 128), lambda i: (0, i, 0)),
        ],
        out_specs=pl.BlockSpec((256, 512), lambda i: (i, 0)),
        out_shape=jax.ShapeDtypeStruct((NPAD, 512), jnp.float32),
    )(x_in, *ys, w0, wf, wb, bias, dinv01)


def _pool(h, oht):
    def body(oh_ref, h_ref, o_ref, acc, cnt):
        i = pl.program_id(0)

        @pl.when(i == 0)
        def _():
            acc[...] = jnp.zeros_like(acc)
            cnt[...] = jnp.zeros_like(cnt)

        oh = oh_ref[...]
        acc[...] += jnp.dot(oh, h_ref[...], preferred_element_type=jnp.float32)
        cnt[...] += jnp.sum(oh, axis=1, keepdims=True)

        @pl.when(i == NB - 1)
        def _():
            o_ref[...] = acc[...] / jnp.maximum(cnt[...][:, 0:1], 1.0)

    return pl.pallas_call(
        body,
        grid=(NB,),
        in_specs=[
            pl.BlockSpec((16, 256), lambda i: (0, i)),
            pl.BlockSpec((256, 512), lambda i: (i, 0)),
        ],
        out_specs=pl.BlockSpec((16, 512), lambda i: (0, 0)),
        out_shape=jax.ShapeDtypeStruct((16, 512), jnp.float32),
        scratch_shapes=[
            pltpu.VMEM((16, 512), jnp.float32),
            pltpu.VMEM((16, 128), jnp.float32),
        ],
    )(oht, h)


def _edge_streams(idx, n_edges):
    extra = EPAD - n_edges
    pad_g = jnp.arange(extra, dtype=jnp.int32) % N_NODES_C
    pad_s = N_NODES_C + (jnp.arange(extra, dtype=jnp.int32) % (NPAD - N_NODES_C))
    shp = (NT, CHUNKS, CL)
    g = jnp.concatenate([idx, pad_g]).reshape(shp)
    s = jnp.concatenate([idx, pad_s]).reshape(shp)
    return g, s


def kernel(x, edge_index, batch, W1f, b1f, W1b, b1b, W2f, b2f, W2b, b2b):
    n, d_in = x.shape
    n_edges = edge_index.shape[1]
    cm = jnp.asarray(_cheb_coeffs(KCH, LAM), jnp.float32)

    src = edge_index[0].astype(jnp.int32)
    dst = edge_index[1].astype(jnp.int32)

    gsrc, ssrc = _edge_streams(src, n_edges)
    gdst, sdst = _edge_streams(dst, n_edges)

    def fused_idx(s_slabs):
        cols = jnp.stack(
            [gdst + (s * NPAD) for s in range(s_slabs)]
            + [gsrc + ((s_slabs + s) * NPAD) for s in range(s_slabs)])
        rows = jnp.stack([ssrc] * s_slabs + [sdst] * s_slabs)
        return cols, rows

    cols1, rows1 = fused_idx(d_in // 128)
    cols2, rows2 = fused_idx(4)

    zeros_stripe = jnp.zeros((STRIPE, 128), jnp.float32)
    spmm1 = _make_spmm(2 * (d_in // 128))
    spmm2 = _make_spmm(8)
    spmm_deg = _make_spmm(2)

    ones_tab = jnp.ones((NPAD, 128), jnp.float32)
    deg2 = spmm_deg(ones_tab, jnp.stack([gsrc, gdst]),
                    jnp.stack([ssrc, sdst]), zeros_stripe)
    dinv01, dinv201 = _dinv_kernel(deg2)

    def prep_w(wf, wb, bf, bb, s_in):
        wpf = jnp.einsum("kj,kio->jio", cm, wf)
        wpb = jnp.einsum("kj,kio->jio", cm, wb)
        w0 = wpf[0] + wpb[0]
        wfs = wpf[1:].reshape(4, s_in, 128, 512)
        wbs = wpb[1:].reshape(4, s_in, 128, 512)
        bias = jnp.broadcast_to((bf + bb).reshape(1, 512), (8, 512))
        return w0, wfs, wbs, bias

    w0_1, wf_1, wb_1, bias_1 = prep_w(W1f, W1b, b1f, b1b, d_in // 128)
    w0_2, wf_2, wb_2, bias_2 = prep_w(W2f, W2b, b2f, b2b, 4)

    x_pad = jnp.pad(x, ((0, NPAD - n), (0, 0)))

    def chain(x2d, cols, rows, s_in, spmm):
        u = _scale_rowmajor2(x2d, dinv01, s_in)
        ys = []
        for j in range(1, KCH):
            y = spmm(u, cols, rows, zeros_stripe)
            ys.append(y.reshape(2 * s_in, NPAD, 128))
            if j < KCH - 1:
                u = _scale_slabmajor2(y, dinv201, s_in)
        return ys

    s1 = d_in // 128
    ys1 = chain(x_pad, cols1, rows1, s1, spmm1)
    h1 = _layer_matmul(x_pad, ys1, w0_1, wf_1, wb_1, bias_1, dinv01, s1)

    ys2 = chain(h1, cols2, rows2, 4, spmm2)
    h2 = _layer_matmul(h1, ys2, w0_2, wf_2, wb_2, bias_2, dinv01, 4)

    batch_pad = jnp.pad(batch.astype(jnp.int32), (0, NPAD - n),
                        constant_values=-1)
    oht = (batch_pad[None, :] == jnp.arange(NGRAPH, dtype=jnp.int32)[:, None]
           ).astype(jnp.float32)
    return _pool(h2, oht)


# speedup vs baseline: 5.7950x; 1.0696x over previous
import functools
import numpy as np
import jax
import jax.numpy as jnp
from jax import lax
from jax.experimental import pallas as pl
from jax.experimental.pallas import tpu as pltpu
from jax.experimental.pallas import tpu_sc as plsc

N_NODES_C = 10000
NPAD = 10240
EPAD = 163840
NT = 16
CHUNKS = 80
CL = 128
GRP = 16
STRIPE = NPAD // NT
NB = NPAD // 256
NGRAPH = 16
LAM = 3.0
KCH = 5


def _cheb_coeffs(k, lam):
    a = 2.0 / lam - 1.0
    b = -2.0 / lam
    cm = np.zeros((k, k), np.float64)
    cm[0, 0] = 1.0
    cm[1, 0] = a
    cm[1, 1] = b
    for i in range(2, k):
        cm[i] = 2.0 * (a * cm[i - 1] + b * np.roll(cm[i - 1], 1)) - cm[i - 2]
    return cm


def _make_spmm(ns):
    mesh = plsc.VectorSubcoreMesh(core_axis_name="c", subcore_axis_name="s")

    @functools.partial(
        pl.kernel,
        mesh=mesh,
        out_type=jax.ShapeDtypeStruct((ns * NPAD, 128), jnp.float32),
        scratch_types=[
            pltpu.VMEM((2, GRP, CL), jnp.int32),
            pltpu.VMEM((2, GRP, CL), jnp.int32),
            pltpu.VMEM((CL, 128), jnp.float32),
            pltpu.VMEM((CL, 128), jnp.float32),
            pltpu.VMEM_SHARED((NPAD, 128), jnp.float32),
            pltpu.SemaphoreType.DMA,
            pltpu.SemaphoreType.DMA,
            pltpu.SemaphoreType.DMA((2,)),
            pltpu.SemaphoreType.DMA((2,)),
        ],
    )
    def k(tab, cols, rows, zstripe, out, colv, rowv, gb0, gb1, acc,
          sem0, sem1, csem, rsem):
        cid = lax.axis_index("c")
        tid = lax.axis_index("s")
        gbufs = (gb0, gb1)
        sems = (sem0, sem1)
        groups = CHUNKS // GRP

        def idx_start(slab, g, ib):
            pltpu.async_copy(
                cols.at[slab, tid, pl.ds(g * GRP, GRP)], colv.at[ib],
                csem.at[ib])
            pltpu.async_copy(
                rows.at[slab, tid, pl.ds(g * GRP, GRP)], rowv.at[ib],
                rsem.at[ib])

        def idx_wait(slab, g, ib):
            pltpu.make_async_copy(
                cols.at[slab, tid, pl.ds(g * GRP, GRP)], colv.at[ib],
                csem.at[ib]).wait()
            pltpu.make_async_copy(
                rows.at[slab, tid, pl.ds(g * GRP, GRP)], rowv.at[ib],
                rsem.at[ib]).wait()

        for si in range(ns // 2):
            slab = 2 * si + cid
            idx_start(slab, 0, 0)
            pltpu.sync_copy(zstripe, acc.at[pl.ds(tid * STRIPE, STRIPE), :])
            plsc.subcore_barrier()

            def group(g, carry):
                ib = lax.rem(g, 2)
                idx_wait(slab, g, ib)
                gn = jnp.minimum(g + 1, groups - 1)
                idx_start(slab, gn, 1 - ib)
                handles = [None, None]
                handles[0] = pltpu.async_copy(
                    tab.at[colv.at[ib, 0]], gb0, sem0)
                for kk in range(GRP):
                    b = kk % 2
                    if kk + 1 < GRP:
                        handles[1 - b] = pltpu.async_copy(
                            tab.at[colv.at[ib, kk + 1]],
                            gbufs[1 - b], sems[1 - b])
                    handles[b].wait()
                    pltpu.sync_copy(gbufs[b], acc.at[rowv.at[ib, kk]],
                                    add=True)
                return carry

            lax.fori_loop(0, groups, group, 0)
            idx_wait(slab, groups - 1, groups % 2)
            plsc.subcore_barrier()
            pltpu.sync_copy(
                acc.at[pl.ds(tid * STRIPE, STRIPE), :],
                out.at[pl.ds(slab * NPAD + tid * STRIPE, STRIPE), :],
            )
            plsc.subcore_barrier()

    return k


def _dinv_scale_kernel(deg2, x, s1):
    def body(d_ref, x_ref, o1, o2, ou):
        deg = d_ref[...]
        dinv = jnp.where(deg > 0, lax.rsqrt(jnp.maximum(deg, 1e-12)), 0.0)
        o1[...] = dinv[None]
        o2[...] = (dinv * dinv)[None]
        for s in range(s1):
            ou[0, s] = x_ref[:, s * 128:(s + 1) * 128] * dinv

    return pl.pallas_call(
        body,
        grid=(2, NB),
        in_specs=[
            pl.BlockSpec((256, 128), lambda d, i: (d * NB + i, 0)),
            pl.BlockSpec((256, s1 * 128), lambda d, i: (i, 0)),
        ],
        out_specs=[
            pl.BlockSpec((1, 256, 128), lambda d, i: (d, i, 0)),
            pl.BlockSpec((1, 256, 128), lambda d, i: (d, i, 0)),
            pl.BlockSpec((1, s1, 256, 128), lambda d, i: (d, 0, i, 0)),
        ],
        out_shape=[
            jax.ShapeDtypeStruct((2, NPAD, 128), jnp.float32),
            jax.ShapeDtypeStruct((2, NPAD, 128), jnp.float32),
            jax.ShapeDtypeStruct((2, s1, NPAD, 128), jnp.float32),
        ],
    )(deg2, x)


def _scale_slabmajor2(y, dscale, s_slabs):
    def body(y_ref, sc_ref, o_ref):
        o_ref[...] = y_ref[...] * sc_ref[0]

    spec = pl.BlockSpec((256, 128), lambda d, s, i: ((d * s_slabs + s) * NB + i, 0))
    return pl.pallas_call(
        body,
        grid=(2, s_slabs, NB),
        in_specs=[
            spec,
            pl.BlockSpec((1, 256, 128), lambda d, s, i: (d, i, 0)),
        ],
        out_specs=spec,
        out_shape=jax.ShapeDtypeStruct((2 * s_slabs * NPAD, 128), jnp.float32),
    )(y, dscale)


def _matmul_core(x_ref, y_refs, w0_ref, wf_ref, wb_ref, b_ref, d_ref, s_in):
    acc = jnp.dot(x_ref[...], w0_ref[...], preferred_element_type=jnp.float32)
    accf = jnp.zeros((256, 512), jnp.float32)
    accb = jnp.zeros((256, 512), jnp.float32)
    for j in range(4):
        for s in range(s_in):
            accf += jnp.dot(y_refs[j][s], wf_ref[j, s],
                            preferred_element_type=jnp.float32)
            accb += jnp.dot(y_refs[j][s_in + s], wb_ref[j, s],
                            preferred_element_type=jnp.float32)
    df = jnp.concatenate([d_ref[0]] * 4, axis=1)
    db = jnp.concatenate([d_ref[1]] * 4, axis=1)
    out = acc + b_ref[0:1, :] + df * accf + db * accb
    return jnp.maximum(out, 0.0), d_ref


def _layer1_matmul(x_in, ys, w0, wf, wb, bias, dinv01, s_in):
    dx = x_in.shape[1]

    def body(x_ref, y0, y1, y2, y3, w0_ref, wf_ref, wb_ref, b_ref,
             d_ref, o_ref, ou_ref):
        h, _ = _matmul_core(x_ref, (y0, y1, y2, y3), w0_ref, wf_ref, wb_ref,
                            b_ref, d_ref, s_in)
        o_ref[...] = h
        for d in range(2):
            dv = d_ref[d]
            for s in range(4):
                ou_ref[d, s] = h[:, s * 128:(s + 1) * 128] * dv

    y_spec = pl.BlockSpec((2 * s_in, 256, 128), lambda i: (0, i, 0))
    return pl.pallas_call(
        body,
        grid=(NB,),
        in_specs=[
            pl.BlockSpec((256, dx), lambda i: (i, 0)),
            y_spec, y_spec, y_spec, y_spec,
            pl.BlockSpec((dx, 512), lambda i: (0, 0)),
            pl.BlockSpec((4, s_in, 128, 512), lambda i: (0, 0, 0, 0)),
            pl.BlockSpec((4, s_in, 128, 512), lambda i: (0, 0, 0, 0)),
            pl.BlockSpec((8, 512), lambda i: (0, 0)),
            pl.BlockSpec((2, 256, 128), lambda i: (0, i, 0)),
        ],
        out_specs=[
            pl.BlockSpec((256, 512), lambda i: (i, 0)),
            pl.BlockSpec((2, 4, 256, 128), lambda i: (0, 0, i, 0)),
        ],
        out_shape=[
            jax.ShapeDtypeStruct((NPAD, 512), jnp.float32),
            jax.ShapeDtypeStruct((2, 4, NPAD, 128), jnp.float32),
        ],
    )(x_in, *ys, w0, wf, wb, bias, dinv01)


def _layer2_matmul_pool(x_in, ys, w0, wf, wb, bias, dinv01, oht, s_in):
    dx = x_in.shape[1]

    def body(x_ref, y0, y1, y2, y3, w0_ref, wf_ref, wb_ref, b_ref,
             d_ref, oh_ref, o_ref, acc, cnt):
        i = pl.program_id(0)

        @pl.when(i == 0)
        def _():
            acc[...] = jnp.zeros_like(acc)
            cnt[...] = jnp.zeros_like(cnt)

        h, _ = _matmul_core(x_ref, (y0, y1, y2, y3), w0_ref, wf_ref, wb_ref,
                            b_ref, d_ref, s_in)
        oh = oh_ref[...]
        acc[...] += jnp.dot(oh, h, preferred_element_type=jnp.float32)
        cnt[...] += jnp.sum(oh, axis=1, keepdims=True)

        @pl.when(i == NB - 1)
        def _():
            o_ref[...] = acc[...] / jnp.maximum(cnt[...][:, 0:1], 1.0)

    y_spec = pl.BlockSpec((2 * s_in, 256, 128), lambda i: (0, i, 0))
    return pl.pallas_call(
        body,
        grid=(NB,),
        in_specs=[
            pl.BlockSpec((256, dx), lambda i: (i, 0)),
            y_spec, y_spec, y_spec, y_spec,
            pl.BlockSpec((dx, 512), lambda i: (0, 0)),
            pl.BlockSpec((4, s_in, 128, 512), lambda i: (0, 0, 0, 0)),
            pl.BlockSpec((4, s_in, 128, 512), lambda i: (0, 0, 0, 0)),
            pl.BlockSpec((8, 512), lambda i: (0, 0)),
            pl.BlockSpec((2, 256, 128), lambda i: (0, i, 0)),
            pl.BlockSpec((16, 256), lambda i: (0, i)),
        ],
        out_specs=pl.BlockSpec((16, 512), lambda i: (0, 0)),
        out_shape=jax.ShapeDtypeStruct((16, 512), jnp.float32),
        scratch_shapes=[
            pltpu.VMEM((16, 512), jnp.float32),
            pltpu.VMEM((16, 128), jnp.float32),
        ],
    )(x_in, *ys, w0, wf, wb, bias, dinv01, oht)


def _edge_streams(idx, n_edges):
    extra = EPAD - n_edges
    pad_g = jnp.arange(extra, dtype=jnp.int32) % N_NODES_C
    pad_s = N_NODES_C + (jnp.arange(extra, dtype=jnp.int32) % (NPAD - N_NODES_C))
    shp = (NT, CHUNKS, CL)
    g = jnp.concatenate([idx, pad_g]).reshape(shp)
    s = jnp.concatenate([idx, pad_s]).reshape(shp)
    return g, s


def kernel(x, edge_index, batch, W1f, b1f, W1b, b1b, W2f, b2f, W2b, b2b):
    n, d_in = x.shape
    n_edges = edge_index.shape[1]
    cm = jnp.asarray(_cheb_coeffs(KCH, LAM), jnp.float32)

    src = edge_index[0].astype(jnp.int32)
    dst = edge_index[1].astype(jnp.int32)

    gsrc, ssrc = _edge_streams(src, n_edges)
    gdst, sdst = _edge_streams(dst, n_edges)

    def fused_idx(s_slabs):
        cols = jnp.stack(
            [gdst + (s * NPAD) for s in range(s_slabs)]
            + [gsrc + ((s_slabs + s) * NPAD) for s in range(s_slabs)])
        rows = jnp.stack([ssrc] * s_slabs + [sdst] * s_slabs)
        return cols, rows

    cols1, rows1 = fused_idx(d_in // 128)
    cols2, rows2 = fused_idx(4)

    zeros_stripe = jnp.zeros((STRIPE, 128), jnp.float32)
    spmm1 = _make_spmm(2 * (d_in // 128))
    spmm2 = _make_spmm(8)
    spmm_deg = _make_spmm(2)

    ones_tab = jnp.ones((NPAD, 128), jnp.float32)
    deg2 = spmm_deg(ones_tab, jnp.stack([gsrc, gdst]),
                    jnp.stack([ssrc, sdst]), zeros_stripe)

    def prep_w(wf, wb, bf, bb, s_in):
        wpf = jnp.einsum("kj,kio->jio", cm, wf)
        wpb = jnp.einsum("kj,kio->jio", cm, wb)
        w0 = wpf[0] + wpb[0]
        wfs = wpf[1:].reshape(4, s_in, 128, 512)
        wbs = wpb[1:].reshape(4, s_in, 128, 512)
        bias = jnp.broadcast_to((bf + bb).reshape(1, 512), (8, 512))
        return w0, wfs, wbs, bias

    w0_1, wf_1, wb_1, bias_1 = prep_w(W1f, W1b, b1f, b1b, d_in // 128)
    w0_2, wf_2, wb_2, bias_2 = prep_w(W2f, W2b, b2f, b2b, 4)

    x_pad = jnp.pad(x, ((0, NPAD - n), (0, 0)))
    s1 = d_in // 128
    dinv01, dinv201, u0_l1 = _dinv_scale_kernel(deg2, x_pad, s1)

    def chain(u0flat, cols, rows, s_in, spmm):
        u = u0flat
        ys = []
        for j in range(1, KCH):
            y = spmm(u, cols, rows, zeros_stripe)
            ys.append(y.reshape(2 * s_in, NPAD, 128))
            if j < KCH - 1:
                u = _scale_slabmajor2(y, dinv201, s_in)
        return ys

    ys1 = chain(u0_l1.reshape(2 * s1 * NPAD, 128), cols1, rows1, s1, spmm1)
    h1, u0_l2 = _layer1_matmul(x_pad, ys1, w0_1, wf_1, wb_1, bias_1,
                               dinv01, s1)

    ys2 = chain(u0_l2.reshape(8 * NPAD, 128), cols2, rows2, 4, spmm2)

    batch_pad = jnp.pad(batch.astype(jnp.int32), (0, NPAD - n),
                        constant_values=-1)
    oht = (batch_pad[None, :] == jnp.arange(NGRAPH, dtype=jnp.int32)[:, None]
           ).astype(jnp.float32)
    return _layer2_matmul_pool(h1, ys2, w0_2, wf_2, wb_2, bias_2,
                               dinv01, oht, 4)
